# Initial kernel scaffold; baseline (speedup 1.0000x reference)
import jax
import jax.numpy as jnp
from jax.experimental import pallas as pl


def kernel(users_ind, items_ind, u_ne_items, u_ne_users, i_ne_users, i_ne_items, u_review_ids, i_review_ids, user_emb, item_emb, review_emb, W1, b1, W2, b2, g1, be1, g2, be2):
    raise NotImplementedError("write your pallas kernel here")


# speedup vs baseline: 1.0192x; 1.0192x over previous
import functools
import math

import jax
import jax.numpy as jnp
from jax.experimental import pallas as pl
from jax.experimental.pallas import tpu as pltpu

B = 4096
K = 32
H = 128
BB = 256


def _ln(x, g, b):
    m = jnp.mean(x, axis=-1, keepdims=True)
    v = jnp.mean((x - m) ** 2, axis=-1, keepdims=True)
    return (x - m) * jax.lax.rsqrt(v + 1e-5) * g + b


def _side(this, nd, ns, rel):
    inv = 1.0 / math.sqrt(H)
    sd = jnp.sum(this[:, None, :] * rel, axis=-1) * inv
    ss = jnp.sum((this * this)[:, None, :] * ns, axis=-1) * inv
    m = jnp.maximum(jnp.max(sd, axis=-1, keepdims=True),
                    jnp.max(ss, axis=-1, keepdims=True))
    ed = jnp.exp(sd - m)
    es = jnp.exp(ss - m)
    z = jnp.sum(ed, axis=-1, keepdims=True) + jnp.sum(es, axis=-1, keepdims=True)
    pref = (jnp.sum(ed[:, :, None] * nd, axis=1)
            + jnp.sum(es[:, :, None] * ns, axis=1)) / z
    return jnp.concatenate([this, pref], axis=-1)


def _transform(x, W1, b1, W2, b2, g1, be1, g2, be2):
    h1 = jnp.maximum(jnp.dot(x, W1, preferred_element_type=jnp.float32) + b1, 0.0)
    h1 = _ln(h1, g1, be1)
    h2 = jnp.maximum(jnp.dot(h1, W2, preferred_element_type=jnp.float32) + b2, 0.0)
    return _ln(h2, g2, be2)


def _gat_block(u_this_ref, i_this_ref, u_nd_ref, u_ns_ref, u_rel_ref,
               i_nd_ref, i_ns_ref, i_rel_ref,
               W1_ref, b1_ref, W2_ref, b2_ref, g1_ref, be1_ref, g2_ref, be2_ref,
               up_ref, ip_ref, rp_ref):
    u_vec = _side(u_this_ref[...], u_nd_ref[...], u_ns_ref[...], u_rel_ref[...])
    i_vec = _side(i_this_ref[...], i_nd_ref[...], i_ns_ref[...], i_rel_ref[...])
    args = (W1_ref[...], b1_ref[...], W2_ref[...], b2_ref[...],
            g1_ref[...], be1_ref[...], g2_ref[...], be2_ref[...])
    up = _transform(u_vec, *args)
    ip = _transform(i_vec, *args)
    up_ref[...] = up
    ip_ref[...] = ip
    rp_ref[...] = up * ip


def _gat_tc(u_this, i_this, u_nd, u_ns, u_rel, i_nd, i_ns, i_rel,
            W1, b1, W2, b2, g1, be1, g2, be2):
    nblk = B // BB
    bspec2 = pl.BlockSpec((BB, H), lambda i: (i, 0))
    bspec3 = pl.BlockSpec((BB, K, H), lambda i: (i, 0, 0))
    wfull = lambda s: pl.BlockSpec(s, lambda i: tuple(0 for _ in s))
    out_shapes = [jax.ShapeDtypeStruct((B, H), jnp.float32)] * 3
    return pl.pallas_call(
        _gat_block,
        grid=(nblk,),
        in_specs=[bspec2, bspec2, bspec3, bspec3, bspec3, bspec3, bspec3, bspec3,
                  wfull((2 * H, H)), wfull((1, H)), wfull((H, H)), wfull((1, H)),
                  wfull((1, H)), wfull((1, H)), wfull((1, H)), wfull((1, H))],
        out_specs=[bspec2, bspec2, bspec2],
        out_shape=out_shapes,
    )(u_this, i_this, u_nd, u_ns, u_rel, i_nd, i_ns, i_rel,
      W1, b1.reshape(1, H), W2, b2.reshape(1, H),
      g1.reshape(1, H), be1.reshape(1, H), g2.reshape(1, H), be2.reshape(1, H))


def kernel(users_ind, items_ind, u_ne_items, u_ne_users, i_ne_users, i_ne_items,
           u_review_ids, i_review_ids, user_emb, item_emb, review_emb,
           W1, b1, W2, b2, g1, be1, g2, be2):
    u_this = jnp.take(user_emb, users_ind, axis=0)
    i_this = jnp.take(item_emb, items_ind, axis=0)
    u_nd = jnp.take(item_emb, u_ne_items, axis=0)
    u_ns = jnp.take(user_emb, u_ne_users, axis=0)
    u_rel = jnp.take(review_emb, u_review_ids, axis=0)
    i_nd = jnp.take(user_emb, i_ne_users, axis=0)
    i_ns = jnp.take(item_emb, i_ne_items, axis=0)
    i_rel = jnp.take(review_emb, i_review_ids, axis=0)
    return tuple(_gat_tc(u_this, i_this, u_nd, u_ns, u_rel, i_nd, i_ns, i_rel,
                         W1, b1, W2, b2, g1, be1, g2, be2))


# speedup vs baseline: 4.2020x; 4.1230x over previous
import functools
import math

import jax
import jax.numpy as jnp
from jax import lax
from jax.experimental import pallas as pl
from jax.experimental.pallas import tpu as pltpu
from jax.experimental.pallas import tpu_sc as plsc

B = 4096
K = 32
H = 128
BB = 256

NC, NS = 2, 16
NW = NC * NS
CH = 128
NCH_BIG = (B * K) // (NW * CH)
NCH_SMALL = B // (NW * CH)


def _sc_gather_body(user_emb, item_emb, review_emb,
                    ix_uthis, ix_ithis, ix_und, ix_uns, ix_url,
                    ix_ind, ix_ins, ix_irl,
                    o_uthis, o_ithis, o_und, o_uns, o_url, o_ind, o_ins, o_irl,
                    idx_v, rows_a, rows_b, sem_a, sem_b):
    w = lax.axis_index("s") * NC + lax.axis_index("c")
    jobs = (
        (user_emb, ix_uthis, o_uthis, NCH_SMALL),
        (item_emb, ix_ithis, o_ithis, NCH_SMALL),
        (item_emb, ix_und, o_und, NCH_BIG),
        (user_emb, ix_uns, o_uns, NCH_BIG),
        (review_emb, ix_url, o_url, NCH_BIG),
        (user_emb, ix_ind, o_ind, NCH_BIG),
        (item_emb, ix_ins, o_ins, NCH_BIG),
        (review_emb, ix_irl, o_irl, NCH_BIG),
    )
    for table, ix_hbm, out_hbm, nch in jobs:
        pltpu.sync_copy(ix_hbm.at[w], idx_v.at[pl.ds(0, nch)])
        base = w * (nch * CH)
        if nch == 1:
            pltpu.async_copy(table.at[idx_v.at[0]], rows_a, sem_a).wait()
            pltpu.sync_copy(rows_a, out_hbm.at[pl.ds(base, CH)])
        else:
            def body(i, _):
                ch = 2 * i
                cp_a = pltpu.async_copy(table.at[idx_v.at[ch]], rows_a, sem_a)
                cp_b = pltpu.async_copy(table.at[idx_v.at[ch + 1]], rows_b, sem_b)
                cp_a.wait()
                pltpu.sync_copy(rows_a, out_hbm.at[pl.ds(base + ch * CH, CH)])
                cp_b.wait()
                pltpu.sync_copy(rows_b, out_hbm.at[pl.ds(base + (ch + 1) * CH, CH)])
                return ()
            lax.fori_loop(0, nch // 2, body, (), unroll=False)


def _sc_gather_all(user_emb, item_emb, review_emb,
                   users_ind, items_ind, u_ne_items, u_ne_users, u_review_ids,
                   i_ne_users, i_ne_items, i_review_ids):
    mesh = plsc.VectorSubcoreMesh(core_axis_name="c", subcore_axis_name="s",
                                  num_cores=NC, num_subcores=NS)
    small = jax.ShapeDtypeStruct((B, H), jnp.float32)
    big = jax.ShapeDtypeStruct((B * K, H), jnp.float32)
    fn = pl.kernel(
        _sc_gather_body,
        out_type=(small, small, big, big, big, big, big, big),
        mesh=mesh,
        scratch_types=[
            pltpu.VMEM((NCH_BIG, CH), jnp.int32),
            pltpu.VMEM((CH, H), jnp.float32),
            pltpu.VMEM((CH, H), jnp.float32),
            pltpu.SemaphoreType.DMA,
            pltpu.SemaphoreType.DMA,
        ],
    )
    rs = lambda a, nch: a.reshape(NW, nch, CH)
    return fn(user_emb, item_emb, review_emb,
              rs(users_ind, NCH_SMALL), rs(items_ind, NCH_SMALL),
              rs(u_ne_items, NCH_BIG), rs(u_ne_users, NCH_BIG),
              rs(u_review_ids, NCH_BIG),
              rs(i_ne_users, NCH_BIG), rs(i_ne_items, NCH_BIG),
              rs(i_review_ids, NCH_BIG))


def _ln(x, g, b):
    m = jnp.mean(x, axis=-1, keepdims=True)
    v = jnp.mean((x - m) ** 2, axis=-1, keepdims=True)
    return (x - m) * jax.lax.rsqrt(v + 1e-5) * g + b


def _side(this, nd, ns, rel):
    inv = 1.0 / math.sqrt(H)
    sd = jnp.sum(this[:, None, :] * rel, axis=-1) * inv
    ss = jnp.sum((this * this)[:, None, :] * ns, axis=-1) * inv
    m = jnp.maximum(jnp.max(sd, axis=-1, keepdims=True),
                    jnp.max(ss, axis=-1, keepdims=True))
    ed = jnp.exp(sd - m)
    es = jnp.exp(ss - m)
    z = jnp.sum(ed, axis=-1, keepdims=True) + jnp.sum(es, axis=-1, keepdims=True)
    pref = (jnp.sum(ed[:, :, None] * nd, axis=1)
            + jnp.sum(es[:, :, None] * ns, axis=1)) / z
    return jnp.concatenate([this, pref], axis=-1)


def _transform(x, W1, b1, W2, b2, g1, be1, g2, be2):
    h1 = jnp.maximum(jnp.dot(x, W1, preferred_element_type=jnp.float32) + b1, 0.0)
    h1 = _ln(h1, g1, be1)
    h2 = jnp.maximum(jnp.dot(h1, W2, preferred_element_type=jnp.float32) + b2, 0.0)
    return _ln(h2, g2, be2)


def _gat_block(u_this_ref, i_this_ref, u_nd_ref, u_ns_ref, u_rel_ref,
               i_nd_ref, i_ns_ref, i_rel_ref,
               W1_ref, b1_ref, W2_ref, b2_ref, g1_ref, be1_ref, g2_ref, be2_ref,
               up_ref, ip_ref, rp_ref):
    u_vec = _side(u_this_ref[...], u_nd_ref[...], u_ns_ref[...], u_rel_ref[...])
    i_vec = _side(i_this_ref[...], i_nd_ref[...], i_ns_ref[...], i_rel_ref[...])
    args = (W1_ref[...], b1_ref[...], W2_ref[...], b2_ref[...],
            g1_ref[...], be1_ref[...], g2_ref[...], be2_ref[...])
    up = _transform(u_vec, *args)
    ip = _transform(i_vec, *args)
    up_ref[...] = up
    ip_ref[...] = ip
    rp_ref[...] = up * ip


def _gat_tc(u_this, i_this, u_nd, u_ns, u_rel, i_nd, i_ns, i_rel,
            W1, b1, W2, b2, g1, be1, g2, be2):
    nblk = B // BB
    bspec2 = pl.BlockSpec((BB, H), lambda i: (i, 0))
    bspec3 = pl.BlockSpec((BB, K, H), lambda i: (i, 0, 0))
    wfull = lambda s: pl.BlockSpec(s, lambda i: tuple(0 for _ in s))
    out_shapes = [jax.ShapeDtypeStruct((B, H), jnp.float32)] * 3
    return pl.pallas_call(
        _gat_block,
        grid=(nblk,),
        in_specs=[bspec2, bspec2, bspec3, bspec3, bspec3, bspec3, bspec3, bspec3,
                  wfull((2 * H, H)), wfull((1, H)), wfull((H, H)), wfull((1, H)),
                  wfull((1, H)), wfull((1, H)), wfull((1, H)), wfull((1, H))],
        out_specs=[bspec2, bspec2, bspec2],
        out_shape=out_shapes,
    )(u_this, i_this, u_nd, u_ns, u_rel, i_nd, i_ns, i_rel,
      W1, b1.reshape(1, H), W2, b2.reshape(1, H),
      g1.reshape(1, H), be1.reshape(1, H), g2.reshape(1, H), be2.reshape(1, H))


def kernel(users_ind, items_ind, u_ne_items, u_ne_users, i_ne_users, i_ne_items,
           u_review_ids, i_review_ids, user_emb, item_emb, review_emb,
           W1, b1, W2, b2, g1, be1, g2, be2):
    (u_this, i_this, u_nd, u_ns, u_rel, i_nd, i_ns, i_rel) = _sc_gather_all(
        user_emb, item_emb, review_emb,
        users_ind, items_ind, u_ne_items, u_ne_users, u_review_ids,
        i_ne_users, i_ne_items, i_review_ids)
    r3 = lambda a: a.reshape(B, K, H)
    return tuple(_gat_tc(u_this, i_this, r3(u_nd), r3(u_ns), r3(u_rel),
                         r3(i_nd), r3(i_ns), r3(i_rel),
                         W1, b1, W2, b2, g1, be1, g2, be2))


# speedup vs baseline: 4.4143x; 1.0505x over previous
import functools
import math

import jax
import jax.numpy as jnp
from jax import lax
from jax.experimental import pallas as pl
from jax.experimental.pallas import tpu as pltpu
from jax.experimental.pallas import tpu_sc as plsc

B = 4096
K = 32
H = 128
BB = 256

NC, NS = 2, 16
NW = NC * NS
CH = 128
def _sc_gather_body(nt_small, nch_big, user_emb, item_emb, review_emb,
                    ix_uthis, ix_ithis, ix_und, ix_uns, ix_url,
                    ix_ind, ix_ins, ix_irl,
                    o_uthis, o_ithis, o_und, o_uns, o_url, o_ind, o_ins, o_irl,
                    idx_v, rows_a, rows_b, sem_a, sem_b):
    w = lax.axis_index("s") * NC + lax.axis_index("c")
    big_jobs = (
        (item_emb, ix_und, o_und),
        (user_emb, ix_uns, o_uns),
        (review_emb, ix_url, o_url),
        (user_emb, ix_ind, o_ind),
        (item_emb, ix_ins, o_ins),
        (review_emb, ix_irl, o_irl),
    )
    for table, ix_hbm, out_hbm in big_jobs:
        pltpu.sync_copy(ix_hbm.at[w], idx_v.at[pl.ds(0, nch_big)])
        base = w * (nch_big * CH)

        def body(i, _):
            ch = 2 * i
            cp_a = pltpu.async_copy(table.at[idx_v.at[ch]], rows_a, sem_a)
            cp_b = pltpu.async_copy(table.at[idx_v.at[ch + 1]], rows_b, sem_b)
            cp_a.wait()
            pltpu.sync_copy(rows_a, out_hbm.at[pl.ds(base + ch * CH, CH)])
            cp_b.wait()
            pltpu.sync_copy(rows_b, out_hbm.at[pl.ds(base + (ch + 1) * CH, CH)])
            return ()
        lax.fori_loop(0, nch_big // 2, body, (), unroll=False)

    wm = jnp.minimum(w, nt_small - 1)
    for table, ix_hbm, out_hbm in ((user_emb, ix_uthis, o_uthis),
                                   (item_emb, ix_ithis, o_ithis)):
        @pl.when(w < nt_small)
        def _():
            pltpu.sync_copy(ix_hbm.at[wm], idx_v.at[pl.ds(0, 1)])
            pltpu.async_copy(table.at[idx_v.at[0]], rows_a, sem_a).wait()
            pltpu.sync_copy(rows_a, out_hbm.at[pl.ds(wm * CH, CH)])


def _sc_gather_all(user_emb, item_emb, review_emb,
                   users_ind, items_ind, u_ne_items, u_ne_users, u_review_ids,
                   i_ne_users, i_ne_items, i_review_ids):
    n = users_ind.shape[0]
    nt_small = n // CH
    nch_big = (n * K) // (NW * CH)
    mesh = plsc.VectorSubcoreMesh(core_axis_name="c", subcore_axis_name="s",
                                  num_cores=NC, num_subcores=NS)
    small = jax.ShapeDtypeStruct((n, H), jnp.float32)
    big = jax.ShapeDtypeStruct((n * K, H), jnp.float32)
    fn = pl.kernel(
        functools.partial(_sc_gather_body, nt_small, nch_big),
        out_type=(small, small, big, big, big, big, big, big),
        mesh=mesh,
        scratch_types=[
            pltpu.VMEM((nch_big, CH), jnp.int32),
            pltpu.VMEM((CH, H), jnp.float32),
            pltpu.VMEM((CH, H), jnp.float32),
            pltpu.SemaphoreType.DMA,
            pltpu.SemaphoreType.DMA,
        ],
    )
    return fn(user_emb, item_emb, review_emb,
              users_ind.reshape(nt_small, 1, CH), items_ind.reshape(nt_small, 1, CH),
              *(a.reshape(NW, nch_big, CH) for a in
                (u_ne_items, u_ne_users, u_review_ids,
                 i_ne_users, i_ne_items, i_review_ids)))


def _ln(x, g, b):
    m = jnp.mean(x, axis=-1, keepdims=True)
    v = jnp.mean((x - m) ** 2, axis=-1, keepdims=True)
    return (x - m) * jax.lax.rsqrt(v + 1e-5) * g + b


def _side(this, nd, ns, rel):
    inv = 1.0 / math.sqrt(H)
    sd = jnp.sum(this[:, None, :] * rel, axis=-1) * inv
    ss = jnp.sum((this * this)[:, None, :] * ns, axis=-1) * inv
    m = jnp.maximum(jnp.max(sd, axis=-1, keepdims=True),
                    jnp.max(ss, axis=-1, keepdims=True))
    ed = jnp.exp(sd - m)
    es = jnp.exp(ss - m)
    z = jnp.sum(ed, axis=-1, keepdims=True) + jnp.sum(es, axis=-1, keepdims=True)
    pref = (jnp.sum(ed[:, :, None] * nd, axis=1)
            + jnp.sum(es[:, :, None] * ns, axis=1)) / z
    return jnp.concatenate([this, pref], axis=-1)


def _transform(x, W1, b1, W2, b2, g1, be1, g2, be2):
    h1 = jnp.maximum(jnp.dot(x, W1, preferred_element_type=jnp.float32) + b1, 0.0)
    h1 = _ln(h1, g1, be1)
    h2 = jnp.maximum(jnp.dot(h1, W2, preferred_element_type=jnp.float32) + b2, 0.0)
    return _ln(h2, g2, be2)


def _gat_block(u_this_ref, i_this_ref, u_nd_ref, u_ns_ref, u_rel_ref,
               i_nd_ref, i_ns_ref, i_rel_ref,
               W1_ref, b1_ref, W2_ref, b2_ref, g1_ref, be1_ref, g2_ref, be2_ref,
               up_ref, ip_ref, rp_ref):
    u_vec = _side(u_this_ref[...], u_nd_ref[...], u_ns_ref[...], u_rel_ref[...])
    i_vec = _side(i_this_ref[...], i_nd_ref[...], i_ns_ref[...], i_rel_ref[...])
    args = (W1_ref[...], b1_ref[...], W2_ref[...], b2_ref[...],
            g1_ref[...], be1_ref[...], g2_ref[...], be2_ref[...])
    up = _transform(u_vec, *args)
    ip = _transform(i_vec, *args)
    up_ref[...] = up
    ip_ref[...] = ip
    rp_ref[...] = up * ip


def _gat_tc(u_this, i_this, u_nd, u_ns, u_rel, i_nd, i_ns, i_rel,
            W1, b1, W2, b2, g1, be1, g2, be2):
    n = u_this.shape[0]
    nblk = n // BB
    bspec2 = pl.BlockSpec((BB, H), lambda i: (i, 0))
    bspec3 = pl.BlockSpec((BB, K, H), lambda i: (i, 0, 0))
    wfull = lambda s: pl.BlockSpec(s, lambda i: tuple(0 for _ in s))
    out_shapes = [jax.ShapeDtypeStruct((n, H), jnp.float32)] * 3
    return pl.pallas_call(
        _gat_block,
        grid=(nblk,),
        in_specs=[bspec2, bspec2, bspec3, bspec3, bspec3, bspec3, bspec3, bspec3,
                  wfull((2 * H, H)), wfull((1, H)), wfull((H, H)), wfull((1, H)),
                  wfull((1, H)), wfull((1, H)), wfull((1, H)), wfull((1, H))],
        out_specs=[bspec2, bspec2, bspec2],
        out_shape=out_shapes,
    )(u_this, i_this, u_nd, u_ns, u_rel, i_nd, i_ns, i_rel,
      W1, b1.reshape(1, H), W2, b2.reshape(1, H),
      g1.reshape(1, H), be1.reshape(1, H), g2.reshape(1, H), be2.reshape(1, H))


NSPLIT = 2


def kernel(users_ind, items_ind, u_ne_items, u_ne_users, i_ne_users, i_ne_items,
           u_review_ids, i_review_ids, user_emb, item_emb, review_emb,
           W1, b1, W2, b2, g1, be1, g2, be2):
    bs = B // NSPLIT
    gathered = []
    for s in range(NSPLIT):
        sl = slice(s * bs, (s + 1) * bs)
        gathered.append(_sc_gather_all(
            user_emb, item_emb, review_emb,
            users_ind[sl], items_ind[sl], u_ne_items[sl], u_ne_users[sl],
            u_review_ids[sl], i_ne_users[sl], i_ne_items[sl], i_review_ids[sl]))
    outs = []
    for s in range(NSPLIT):
        (u_this, i_this, u_nd, u_ns, u_rel, i_nd, i_ns, i_rel) = gathered[s]
        r3 = lambda a: a.reshape(bs, K, H)
        outs.append(_gat_tc(u_this, i_this, r3(u_nd), r3(u_ns), r3(u_rel),
                            r3(i_nd), r3(i_ns), r3(i_rel),
                            W1, b1, W2, b2, g1, be1, g2, be2))
    return tuple(jnp.concatenate([o[j] for o in outs], axis=0) for j in range(3))
